# baseline (speedup 1.0000x reference)
import functools

import jax
import jax.numpy as jnp
from jax import lax
from jax.experimental import pallas as pl
from jax.experimental.pallas import tpu as pltpu
from jax.experimental.pallas import tpu_sc as plsc

_LANES = 128


def _sc_geometry():
    try:
        info = plsc.get_sparse_core_info()
        return info.num_cores, info.num_subcores
    except Exception:
        return 2, 16


@functools.cache
def _build(S, BB, NF, D):
    NC, NS = _sc_geometry()
    NW = NC * NS
    NBLK = S * BB
    BW = NBLK // NW
    RW = BW * NF
    DHI = D // 8
    assert NBLK % NW == 0 and BW % 2 == 0 and D % 16 == 0

    mesh = plsc.VectorSubcoreMesh(core_axis_name="c", subcore_axis_name="s")

    @functools.partial(
        pl.kernel,
        mesh=mesh,
        compiler_params=pltpu.CompilerParams(
            use_tc_tiling_on_sc=False, needs_layout_passes=False),
        out_type=jax.ShapeDtypeStruct((S, DHI, BB, 8, _LANES), jnp.float32),
        scratch_types=[
            pltpu.VMEM((RW, _LANES), jnp.int32),
            pltpu.VMEM((NF * _LANES, D), jnp.float32),
            pltpu.VMEM((NF * _LANES, D), jnp.float32),
            pltpu.VMEM((DHI, 8, _LANES), jnp.float32),
            pltpu.VMEM((DHI, 8, _LANES), jnp.float32),
            pltpu.SemaphoreType.DMA,
            pltpu.SemaphoreType.DMA,
            pltpu.SemaphoreType.DMA,
            pltpu.SemaphoreType.DMA,
        ],
    )
    def run(idx_hbm, table_hbm, out_hbm, idx_v, rows0, rows1, outv0, outv1,
            gsem0, gsem1, osem0, osem1):
        wid = lax.axis_index("s") * NC + lax.axis_index("c")
        blk0 = wid * BW

        r0 = pl.multiple_of(wid * RW, RW)
        pltpu.sync_copy(idx_hbm.at[pl.ds(r0, RW)], idx_v)

        @plsc.parallel_loop(0, RW, unroll=2)
        def _remap(i):
            for u in range(_LANES // 16):
                sl = pl.ds(u * 16, 16)
                r = idx_v[i, sl]
                idx_v[i, sl] = ((r & -256) | ((r & 127) << 1)
                                | ((r >> 7) & 1))

        iota = lax.iota(jnp.int32, 16)
        dlo = iota & 7
        dhi_q = [(iota >> 3) + 2 * q for q in range(D // 16)]

        def fire_gathers(k, rows, gsem):
            descs = []
            for f in range(NF):
                descs.append(pltpu.async_copy(
                    table_hbm.at[idx_v.at[k * NF + f]],
                    rows.at[pl.ds(f * _LANES, _LANES)],
                    gsem,
                ))
            return descs

        def drain_gathers(k, rows, gsem):
            pltpu.make_async_copy(
                table_hbm.at[pl.ds(0, NF * _LANES)], rows, gsem).wait()

        def compute(rows, outv):
            @plsc.parallel_loop(0, _LANES, unroll=8)
            def _tok(t):
                tv = lax.broadcast(t, (16,))
                for q in range(D // 16):
                    sl = pl.ds(q * 16, 16)
                    acc = rows[t, sl]
                    for f in range(1, NF):
                        acc = acc + rows[f * _LANES + t, sl]
                    plsc.store_scatter(outv, [dhi_q[q], dlo, tv], acc)

        def write_out(k, outv, osem):
            blk = blk0 + k
            s = blk // BB
            j = blk % BB
            return pltpu.async_copy(outv, out_hbm.at[s, :, j], osem)

        def drain_out(k, outv, osem):
            pltpu.make_async_copy(
                out_hbm.at[0, 0], outv, osem).wait()

        fire_gathers(0, rows0, gsem0)

        def body(m, carry):
            bufs = ((rows0, outv0, gsem0, osem0),
                    (rows1, outv1, gsem1, osem1))
            for p in range(2):
                rows, outv, gsem, osem = bufs[p]
                k = 2 * m + p
                nrows, _, ngsem, _ = bufs[1 - p]
                if p == 0:
                    fire_gathers(k + 1, nrows, ngsem)
                else:
                    @pl.when(m < BW // 2 - 1)
                    def _():
                        fire_gathers(k + 1, nrows, ngsem)
                drain_gathers(k, rows, gsem)

                @pl.when(m > 0)
                def _():
                    drain_out(k - 2, outv, osem)
                compute(rows, outv)
                write_out(k, outv, osem)
            return carry

        lax.fori_loop(0, BW // 2, body, 0)
        drain_out(BW - 2, outv0, osem0)
        drain_out(BW - 1, outv1, osem1)

    return run


@functools.cache
def _transpose_table(V, D, C=8192):
    CW = 4 * D
    assert C % CW == 0
    G = (V + C - 1) // C

    def body(t_ref, o_ref):
        for k in range(C // CW):
            a = t_ref[:, k * CW: k * CW + 2 * D]
            b = t_ref[:, k * CW + 2 * D: (k + 1) * CW]
            xx = jnp.concatenate([a, b], axis=0)
            o_ref[pl.ds(k * 2 * D, 2 * D), :] = xx.T

    return pl.pallas_call(
        body,
        grid=(G,),
        in_specs=[pl.BlockSpec((D, C), lambda i: (0, i))],
        out_specs=pl.BlockSpec((C // 2, 2 * D), lambda i: (i, 0)),
        out_shape=jax.ShapeDtypeStruct((G * C // 2, 2 * D), jnp.float32),
    )


def kernel(src_tokens, table):
    bsz, seqlen, nf = src_tokens.shape
    _, D = table.shape
    bb = bsz // _LANES
    idx = (src_tokens.astype(jnp.int32)
           .reshape(bb, _LANES, seqlen, nf)
           .transpose(2, 0, 3, 1)
           .reshape(seqlen * bb * nf, _LANES))
    V = table.shape[0]
    tab_rm = _transpose_table(V, D)(table.T)
    Vp = tab_rm.shape[0] * 2
    tab_rm = tab_rm.reshape(Vp, D)
    out5 = _build(seqlen, bb, nf, D)(idx, tab_rm)
    return (out5.transpose(2, 4, 0, 1, 3).reshape(bsz, seqlen, D))


# speedup vs baseline: 1.0735x; 1.0735x over previous
import functools

import jax
import jax.numpy as jnp
from jax import lax
from jax.experimental import pallas as pl
from jax.experimental.pallas import tpu as pltpu
from jax.experimental.pallas import tpu_sc as plsc

_LANES = 128


def _sc_geometry():
    try:
        info = plsc.get_sparse_core_info()
        return info.num_cores, info.num_subcores
    except Exception:
        return 2, 16


@functools.cache
def _build(S, BB, NF, D):
    NC, NS = _sc_geometry()
    NW = NC * NS
    NBLK = S * BB
    BW = NBLK // NW
    RW = BW * NF
    DHI = D // 8
    assert NBLK % NW == 0 and BW % 2 == 0 and D % 16 == 0

    mesh = plsc.VectorSubcoreMesh(core_axis_name="c", subcore_axis_name="s")

    @functools.partial(
        pl.kernel,
        mesh=mesh,
        compiler_params=pltpu.CompilerParams(
            use_tc_tiling_on_sc=False, needs_layout_passes=False),
        out_type=jax.ShapeDtypeStruct((S, DHI, BB, 8, _LANES), jnp.float32),
        scratch_types=[
            pltpu.VMEM((RW, _LANES), jnp.int32),
            pltpu.VMEM((NF * _LANES, D), jnp.float32),
            pltpu.VMEM((NF * _LANES, D), jnp.float32),
            pltpu.VMEM((DHI, 8, _LANES), jnp.float32),
            pltpu.VMEM((DHI, 8, _LANES), jnp.float32),
            pltpu.SemaphoreType.DMA,
            pltpu.SemaphoreType.DMA,
            pltpu.SemaphoreType.DMA,
            pltpu.SemaphoreType.DMA,
        ],
    )
    def run(idx_hbm, table_hbm, out_hbm, idx_v, rows0, rows1, outv0, outv1,
            gsem0, gsem1, osem0, osem1):
        wid = lax.axis_index("s") * NC + lax.axis_index("c")
        blk0 = wid * BW

        r0 = pl.multiple_of(wid * RW, RW)
        pltpu.sync_copy(idx_hbm.at[pl.ds(r0, RW)], idx_v)

        @plsc.parallel_loop(0, RW, unroll=2)
        def _remap(i):
            for u in range(_LANES // 16):
                sl = pl.ds(u * 16, 16)
                r = idx_v[i, sl]
                idx_v[i, sl] = ((r & -256) | ((r & 127) << 1)
                                | ((r >> 7) & 1))

        iota = lax.iota(jnp.int32, 16)
        dlo = iota & 7
        dhi_q = [(iota >> 3) + 2 * q for q in range(D // 16)]

        def fire_gathers(k, rows, gsem):
            descs = []
            for f in range(NF):
                descs.append(pltpu.async_copy(
                    table_hbm.at[idx_v.at[k * NF + f]],
                    rows.at[pl.ds(f * _LANES, _LANES)],
                    gsem,
                ))
            return descs

        def drain_gathers(k, rows, gsem):
            pltpu.make_async_copy(
                table_hbm.at[pl.ds(0, NF * _LANES)], rows, gsem).wait()

        def compute(rows, outv):
            @plsc.parallel_loop(0, _LANES, unroll=4)
            def _tok(t):
                tv = lax.broadcast(t, (16,))
                for q in range(D // 16):
                    sl = pl.ds(q * 16, 16)
                    acc = rows[t, sl]
                    for f in range(1, NF):
                        acc = acc + rows[f * _LANES + t, sl]
                    plsc.store_scatter(outv, [dhi_q[q], dlo, tv], acc)

        def write_out(k, outv, osem):
            blk = blk0 + k
            s = blk // BB
            j = blk % BB
            return pltpu.async_copy(outv, out_hbm.at[s, :, j], osem)

        def drain_out(k, outv, osem):
            pltpu.make_async_copy(
                out_hbm.at[0, 0], outv, osem).wait()

        fire_gathers(0, rows0, gsem0)

        def body(m, carry):
            bufs = ((rows0, outv0, gsem0, osem0),
                    (rows1, outv1, gsem1, osem1))
            for p in range(2):
                rows, outv, gsem, osem = bufs[p]
                k = 2 * m + p
                nrows, _, ngsem, _ = bufs[1 - p]
                if p == 0:
                    fire_gathers(k + 1, nrows, ngsem)
                else:
                    @pl.when(m < BW // 2 - 1)
                    def _():
                        fire_gathers(k + 1, nrows, ngsem)
                drain_gathers(k, rows, gsem)

                @pl.when(m > 0)
                def _():
                    drain_out(k - 2, outv, osem)
                compute(rows, outv)
                write_out(k, outv, osem)
            return carry

        lax.fori_loop(0, BW // 2, body, 0)
        drain_out(BW - 2, outv0, osem0)
        drain_out(BW - 1, outv1, osem1)

    return run


@functools.cache
def _transpose_table(V, D, C=16384):
    CW = 4 * D
    assert C % CW == 0
    G = (V + C - 1) // C

    def body(t_ref, o_ref):
        for k in range(C // CW):
            a = t_ref[:, k * CW: k * CW + 2 * D]
            b = t_ref[:, k * CW + 2 * D: (k + 1) * CW]
            xx = jnp.concatenate([a, b], axis=0)
            o_ref[pl.ds(k * 2 * D, 2 * D), :] = xx.T

    return pl.pallas_call(
        body,
        grid=(G,),
        in_specs=[pl.BlockSpec((D, C), lambda i: (0, i))],
        out_specs=pl.BlockSpec((C // 2, 2 * D), lambda i: (i, 0)),
        out_shape=jax.ShapeDtypeStruct((G * C // 2, 2 * D), jnp.float32),
    )


def kernel(src_tokens, table):
    bsz, seqlen, nf = src_tokens.shape
    _, D = table.shape
    bb = bsz // _LANES
    idx = (src_tokens.astype(jnp.int32)
           .reshape(bb, _LANES, seqlen, nf)
           .transpose(2, 0, 3, 1)
           .reshape(seqlen * bb * nf, _LANES))
    V = table.shape[0]
    tab_rm = _transpose_table(V, D)(table.T)
    Vp = tab_rm.shape[0] * 2
    tab_rm = tab_rm.reshape(Vp, D)
    out5 = _build(seqlen, bb, nf, D)(idx, tab_rm)
    return (out5.transpose(2, 4, 0, 1, 3).reshape(bsz, seqlen, D))


# speedup vs baseline: 1.0877x; 1.0132x over previous
import functools

import jax
import jax.numpy as jnp
from jax import lax
from jax.experimental import pallas as pl
from jax.experimental.pallas import tpu as pltpu
from jax.experimental.pallas import tpu_sc as plsc

_LANES = 128


def _sc_geometry():
    try:
        info = plsc.get_sparse_core_info()
        return info.num_cores, info.num_subcores
    except Exception:
        return 2, 16


@functools.cache
def _build(S, BB, NF, D):
    NC, NS = _sc_geometry()
    NW = NC * NS
    NBLK = S * BB
    BW = NBLK // NW
    RW = BW * NF
    DHI = D // 8
    assert NBLK % NW == 0 and BW % 2 == 0 and D % 16 == 0

    mesh = plsc.VectorSubcoreMesh(core_axis_name="c", subcore_axis_name="s")

    @functools.partial(
        pl.kernel,
        mesh=mesh,
        compiler_params=pltpu.CompilerParams(
            use_tc_tiling_on_sc=False, needs_layout_passes=False),
        out_type=jax.ShapeDtypeStruct((S, DHI, BB, 8, _LANES), jnp.float32),
        scratch_types=[
            pltpu.VMEM((RW, _LANES), jnp.int32),
            pltpu.VMEM((NF * _LANES, D), jnp.float32),
            pltpu.VMEM((NF * _LANES, D), jnp.float32),
            pltpu.VMEM((DHI, 8, _LANES), jnp.float32),
            pltpu.VMEM((DHI, 8, _LANES), jnp.float32),
            pltpu.SemaphoreType.DMA,
            pltpu.SemaphoreType.DMA,
            pltpu.SemaphoreType.DMA,
            pltpu.SemaphoreType.DMA,
        ],
    )
    def run(idx_hbm, table_hbm, out_hbm, idx_v, rows0, rows1, outv0, outv1,
            gsem0, gsem1, osem0, osem1):
        wid = lax.axis_index("s") * NC + lax.axis_index("c")
        blk0 = wid * BW

        r0 = pl.multiple_of(wid * RW, RW)
        pltpu.sync_copy(idx_hbm.at[pl.ds(r0, RW)], idx_v)

        @plsc.parallel_loop(0, RW, unroll=2)
        def _remap(i):
            for u in range(_LANES // 16):
                sl = pl.ds(u * 16, 16)
                r = idx_v[i, sl]
                idx_v[i, sl] = ((r & -256) | ((r & 127) << 1)
                                | ((r >> 7) & 1))

        iota = lax.iota(jnp.int32, 16)
        dlo = iota & 7
        dhi_q = [(iota >> 3) + 2 * q for q in range(D // 16)]

        def fire_gathers(k, rows, gsem):
            descs = []
            for f in range(NF):
                descs.append(pltpu.async_copy(
                    table_hbm.at[idx_v.at[k * NF + f]],
                    rows.at[pl.ds(f * _LANES, _LANES)],
                    gsem,
                ))
            return descs

        def drain_gathers(k, rows, gsem):
            pltpu.make_async_copy(
                table_hbm.at[pl.ds(0, NF * _LANES)], rows, gsem).wait()

        def compute(rows, outv):
            @plsc.parallel_loop(0, _LANES, unroll=4)
            def _tok(t):
                tv = lax.broadcast(t, (16,))
                for q in range(D // 16):
                    sl = pl.ds(q * 16, 16)
                    acc = rows[t, sl]
                    for f in range(1, NF):
                        acc = acc + rows[f * _LANES + t, sl]
                    plsc.store_scatter(outv, [dhi_q[q], dlo, tv], acc)

        def write_out(k, outv, osem):
            blk = blk0 + k
            s = blk // BB
            j = blk % BB
            return pltpu.async_copy(outv, out_hbm.at[s, :, j], osem)

        def drain_out(k, outv, osem):
            pltpu.make_async_copy(
                out_hbm.at[0, 0], outv, osem).wait()

        fire_gathers(0, rows0, gsem0)

        def body(m, carry):
            bufs = ((rows0, outv0, gsem0, osem0),
                    (rows1, outv1, gsem1, osem1))
            for p in range(2):
                rows, outv, gsem, osem = bufs[p]
                k = 2 * m + p
                nrows, _, ngsem, _ = bufs[1 - p]
                if p == 0:
                    fire_gathers(k + 1, nrows, ngsem)
                else:
                    @pl.when(m < BW // 2 - 1)
                    def _():
                        fire_gathers(k + 1, nrows, ngsem)
                drain_gathers(k, rows, gsem)

                @pl.when(m > 0)
                def _():
                    drain_out(k - 2, outv, osem)
                compute(rows, outv)
                write_out(k, outv, osem)
            return carry

        lax.fori_loop(0, BW // 2, body, 0)
        drain_out(BW - 2, outv0, osem0)
        drain_out(BW - 1, outv1, osem1)

    return run


@functools.cache
def _transpose_table(V, D, C=32768):
    CW = 4 * D
    assert C % CW == 0
    G = (V + C - 1) // C

    def body(t_ref, o_ref):
        for k in range(C // CW):
            a = t_ref[:, k * CW: k * CW + 2 * D]
            b = t_ref[:, k * CW + 2 * D: (k + 1) * CW]
            xx = jnp.concatenate([a, b], axis=0)
            o_ref[pl.ds(k * 2 * D, 2 * D), :] = xx.T

    return pl.pallas_call(
        body,
        grid=(G,),
        in_specs=[pl.BlockSpec((D, C), lambda i: (0, i))],
        out_specs=pl.BlockSpec((C // 2, 2 * D), lambda i: (i, 0)),
        out_shape=jax.ShapeDtypeStruct((G * C // 2, 2 * D), jnp.float32),
    )


def kernel(src_tokens, table):
    bsz, seqlen, nf = src_tokens.shape
    _, D = table.shape
    bb = bsz // _LANES
    idx = (src_tokens.astype(jnp.int32)
           .reshape(bb, _LANES, seqlen, nf)
           .transpose(2, 0, 3, 1)
           .reshape(seqlen * bb * nf, _LANES))
    V = table.shape[0]
    tab_rm = _transpose_table(V, D)(table.T)
    Vp = tab_rm.shape[0] * 2
    tab_rm = tab_rm.reshape(Vp, D)
    out5 = _build(seqlen, bb, nf, D)(idx, tab_rm)
    return (out5.transpose(2, 4, 0, 1, 3).reshape(bsz, seqlen, D))
